# baseline (speedup 1.0000x reference)
import jax
import jax.numpy as jnp
from jax import lax
from jax.experimental import pallas as pl
from jax.experimental.pallas import tpu as pltpu
from jax.experimental.pallas import tpu_sc as plsc

_L = 16
_NC = 2
_NS = 16
_NW = _NC * _NS
_B = 16384
_D = 64
_BPW = _B // _NW
_NBUF = 7


def _sc_body(uidx_hbm, iidx_hbm, ut_hbm, it_hbm, out_hbm,
             uidx_v, iidx_v, ublk, iblk, out_v, usem, isem):
    wid = lax.axis_index("s") * _NC + lax.axis_index("c")
    base = wid * _BPW

    pltpu.sync_copy(uidx_hbm.at[pl.ds(base, _BPW)], uidx_v)
    pltpu.sync_copy(iidx_hbm.at[pl.ds(base, _BPW)], iidx_v)

    lane = lax.iota(jnp.int32, _L)

    def group_body(g, _):
        g16 = g * _L
        uvec = uidx_v[pl.ds(g16, _L)]
        ivec = iidx_v[pl.ds(g16, _L)]

        def fire(j):
            slot = j % _NBUF
            ru = uvec[j]
            ri = ivec[j]
            cu0 = pl.multiple_of(lax.shift_right_logical(ru, 7) * 128, 128)
            ci0 = pl.multiple_of(lax.shift_right_logical(ri, 7) * 128, 128)
            return (
                pltpu.async_copy(ut_hbm.at[:, pl.ds(cu0, 128)],
                                 ublk.at[slot], usem.at[slot]),
                pltpu.async_copy(it_hbm.at[:, pl.ds(ci0, 128)],
                                 iblk.at[slot], isem.at[slot]),
            )

        pending = [fire(j) for j in range(_NBUF)]
        accv = jnp.zeros((_L,), jnp.float32)
        for j in range(_L):
            slot = j % _NBUF
            cpu, cpi = pending[slot]
            cpu.wait()
            cpi.wait()
            cu = jnp.bitwise_and(uvec[j], 127)
            ci = jnp.bitwise_and(ivec[j], 127)
            owu = jnp.bitwise_and(cu, 127 - 15)
            owi = jnp.bitwise_and(ci, 127 - 15)
            lu = jnp.full((_L,), 0, jnp.int32) + jnp.bitwise_and(cu, 15)
            li = jnp.full((_L,), 0, jnp.int32) + jnp.bitwise_and(ci, 15)

            def dim_body(k, a):
                for dd in range(_L):
                    d = k * _L + dd
                    uwin = ublk[slot, d, pl.ds(owu, _L)]
                    iwin = iblk[slot, d, pl.ds(owi, _L)]
                    a = a + jnp.take(uwin, lu, axis=0) * jnp.take(iwin, li, axis=0)
                return a

            acc = lax.fori_loop(0, _D // _L, dim_body,
                                jnp.zeros((_L,), jnp.float32))
            accv = jnp.where(lane == j, acc, accv)
            if j + _NBUF < _L:
                pending[slot] = fire(j + _NBUF)
        out_v[pl.ds(g16, _L)] = accv
        return _

    lax.fori_loop(0, _BPW // _L, group_body, 0)

    pltpu.sync_copy(out_v, out_hbm.at[pl.ds(base, _BPW)])


@jax.jit
def _run(user_indices, item_indices, ut, it):
    mesh = plsc.VectorSubcoreMesh(core_axis_name="c", subcore_axis_name="s")
    f = pl.kernel(
        _sc_body,
        mesh=mesh,
        out_type=jax.ShapeDtypeStruct((_B,), jnp.float32),
        scratch_types=[
            pltpu.VMEM((_BPW,), jnp.int32),
            pltpu.VMEM((_BPW,), jnp.int32),
            pltpu.VMEM((_NBUF, _D, 128), jnp.float32),
            pltpu.VMEM((_NBUF, _D, 128), jnp.float32),
            pltpu.VMEM((_BPW,), jnp.float32),
            pltpu.SemaphoreType.DMA((_NBUF,)),
            pltpu.SemaphoreType.DMA((_NBUF,)),
        ],
    )
    return f(user_indices, item_indices, ut, it)


def kernel(user_indices, item_indices, user_table, item_table):
    return _run(user_indices.astype(jnp.int32), item_indices.astype(jnp.int32),
                user_table.T, item_table.T)
